# baseline (speedup 1.0000x reference)
import jax
import jax.numpy as jnp
from jax import lax
from jax.experimental import pallas as pl
from jax.experimental.pallas import tpu as pltpu
from jax.experimental.pallas import tpu_sc as plsc

_B = 16384
_NC = 2
_NS = 16
_NW = _NC * _NS
_BPW = _B // _NW
_CHUNK = 128
_NJ = _BPW // _CHUNK


def _gather_body(rx_h, ry_h, rz_h, tx_h, ty_h, tz_h, f_h, px_h, py_h, idx_hbm,
                 r_out, t_out, ff_out, pps_out,
                 idx_v, r_v, t_v, f_v, pp_v, sem, sem_out):
    wid = lax.axis_index("s") * _NC + lax.axis_index("c")
    base = wid * _BPW

    stage = [pltpu.async_copy(idx_hbm.at[pl.ds(base + j * _CHUNK, _CHUNK)],
                              idx_v.at[j], sem)
             for j in range(_NJ)]
    for c in stage:
        c.wait()

    for j in range(_NJ):
        o = j * _CHUNK
        ids = idx_v.at[j]
        for c, (src, cat) in enumerate(((rx_h, r_v), (ry_h, r_v), (rz_h, r_v),
                                        (tx_h, t_v), (ty_h, t_v), (tz_h, t_v),
                                        (px_h, pp_v), (py_h, pp_v))):
            cc = c % 3 if c < 6 else c - 6
            pltpu.async_copy(src.at[ids],
                             cat.at[pl.ds(cc * _BPW + o, _CHUNK)], sem)
        pltpu.async_copy(f_h.at[ids], f_v.at[pl.ds(o, _CHUNK)], sem)

    pltpu.make_async_copy(f_h.at[pl.ds(0, _BPW)], f_v, sem).wait()
    pltpu.make_async_copy(px_h.at[pl.ds(0, _BPW * 2)], pp_v, sem).wait()
    pltpu.make_async_copy(rx_h.at[pl.ds(0, _BPW * 3)], r_v, sem).wait()
    pltpu.make_async_copy(tx_h.at[pl.ds(0, _BPW * 3)], t_v, sem).wait()

    out = []
    for c in range(3):
        out.append(pltpu.async_copy(r_v.at[pl.ds(c * _BPW, _BPW)],
                                    r_out.at[pl.ds(c * _B + base, _BPW)],
                                    sem_out))
        out.append(pltpu.async_copy(t_v.at[pl.ds(c * _BPW, _BPW)],
                                    t_out.at[pl.ds(c * _B + base, _BPW)],
                                    sem_out))
    for c in range(2):
        out.append(pltpu.async_copy(pp_v.at[pl.ds(c * _BPW, _BPW)],
                                    pps_out.at[pl.ds(c * _B + base, _BPW)],
                                    sem_out))
    out.append(pltpu.async_copy(f_v, ff_out.at[pl.ds(base, _BPW)], sem_out))
    for c in out:
        c.wait()


def kernel(rotvecs, translations, f, pp, camera_idxs):
    idxf = camera_idxs.astype(jnp.int32)
    mesh = plsc.VectorSubcoreMesh(core_axis_name="c", subcore_axis_name="s")
    run = pl.kernel(
        _gather_body,
        out_type=(
            jax.ShapeDtypeStruct((_B * 3,), jnp.float32),
            jax.ShapeDtypeStruct((_B * 3,), jnp.float32),
            jax.ShapeDtypeStruct((_B,), jnp.float32),
            jax.ShapeDtypeStruct((_B * 2,), jnp.float32),
        ),
        mesh=mesh,
        scratch_types=[
            pltpu.VMEM((_NJ, _CHUNK), jnp.int32),
            pltpu.VMEM((_BPW * 3,), jnp.float32),
            pltpu.VMEM((_BPW * 3,), jnp.float32),
            pltpu.VMEM((_BPW,), jnp.float32),
            pltpu.VMEM((_BPW * 2,), jnp.float32),
            pltpu.SemaphoreType.DMA,
            pltpu.SemaphoreType.DMA,
        ],
    )
    rT, tT, ff, pT = run(rotvecs[:, 0], rotvecs[:, 1], rotvecs[:, 2],
                         translations[:, 0], translations[:, 1],
                         translations[:, 2], f, pp[:, 0], pp[:, 1], idxf)
    r = rT.reshape(3, _B).T
    t = tT.reshape(3, _B).T
    pps = pT.reshape(2, _B).T
    return (r, t, ff, pps)


# speedup vs baseline: 1.2143x; 1.2143x over previous
import jax
import jax.numpy as jnp
from jax import lax
from jax.experimental import pallas as pl
from jax.experimental.pallas import tpu as pltpu
from jax.experimental.pallas import tpu_sc as plsc

_N = 100000
_B = 16384
_NC = 2
_NS = 16
_NW = _NC * _NS
_BPW = _B // _NW
_CHUNK = 128
_L = 16
_NG = _CHUNK // _L
_NJ = _BPW // _CHUNK


def _gather_body(rot_hbm, tr_hbm, f_hbm, pp_hbm, idx_hbm,
                 r_out, t_out, ff_out, pps_out,
                 idx_v, iy, iz, r_v, t_v, f_v, pp_v, sem, sem_out):
    wid = lax.axis_index("s") * _NC + lax.axis_index("c")
    base = wid * _BPW

    stage = [pltpu.async_copy(idx_hbm.at[pl.ds(base + j * _CHUNK, _CHUNK)],
                              idx_v.at[j], sem)
             for j in range(_NJ)]
    for c in stage:
        c.wait()
    for j in range(_NJ):
        for g in range(_NG):
            s = pl.ds(g * _L, _L)
            v = idx_v[j, s]
            iy[j, s] = v + _N
            iz[j, s] = v + 2 * _N

    for j in range(_NJ):
        o = j * _CHUNK
        for ilist, cat, c in ((idx_v, r_v, 0), (iy, r_v, 1), (iz, r_v, 2)):
            pltpu.async_copy(rot_hbm.at[ilist.at[j]],
                             cat.at[pl.ds(c * _BPW + o, _CHUNK)], sem)
        for ilist, cat, c in ((idx_v, t_v, 0), (iy, t_v, 1), (iz, t_v, 2)):
            pltpu.async_copy(tr_hbm.at[ilist.at[j]],
                             cat.at[pl.ds(c * _BPW + o, _CHUNK)], sem)
        for ilist, c in ((idx_v, 0), (iy, 1)):
            pltpu.async_copy(pp_hbm.at[ilist.at[j]],
                             pp_v.at[pl.ds(c * _BPW + o, _CHUNK)], sem)
        pltpu.async_copy(f_hbm.at[idx_v.at[j]],
                         f_v.at[pl.ds(o, _CHUNK)], sem)

    pltpu.make_async_copy(rot_hbm.at[pl.ds(0, _BPW * 3)], r_v, sem).wait()
    pltpu.make_async_copy(tr_hbm.at[pl.ds(0, _BPW * 3)], t_v, sem).wait()
    pltpu.make_async_copy(f_hbm.at[pl.ds(0, _BPW)], f_v, sem).wait()
    pltpu.make_async_copy(pp_hbm.at[pl.ds(0, _BPW * 2)], pp_v, sem).wait()

    out = []
    for c in range(3):
        out.append(pltpu.async_copy(r_v.at[pl.ds(c * _BPW, _BPW)],
                                    r_out.at[pl.ds(c * _B + base, _BPW)],
                                    sem_out))
        out.append(pltpu.async_copy(t_v.at[pl.ds(c * _BPW, _BPW)],
                                    t_out.at[pl.ds(c * _B + base, _BPW)],
                                    sem_out))
    for c in range(2):
        out.append(pltpu.async_copy(pp_v.at[pl.ds(c * _BPW, _BPW)],
                                    pps_out.at[pl.ds(c * _B + base, _BPW)],
                                    sem_out))
    out.append(pltpu.async_copy(f_v, ff_out.at[pl.ds(base, _BPW)], sem_out))
    for c in out:
        c.wait()


def kernel(rotvecs, translations, f, pp, camera_idxs):
    idxf = camera_idxs.astype(jnp.int32)
    mesh = plsc.VectorSubcoreMesh(core_axis_name="c", subcore_axis_name="s")
    run = pl.kernel(
        _gather_body,
        out_type=(
            jax.ShapeDtypeStruct((_B * 3,), jnp.float32),
            jax.ShapeDtypeStruct((_B * 3,), jnp.float32),
            jax.ShapeDtypeStruct((_B,), jnp.float32),
            jax.ShapeDtypeStruct((_B * 2,), jnp.float32),
        ),
        mesh=mesh,
        scratch_types=[
            pltpu.VMEM((_NJ, _CHUNK), jnp.int32),
            pltpu.VMEM((_NJ, _CHUNK), jnp.int32),
            pltpu.VMEM((_NJ, _CHUNK), jnp.int32),
            pltpu.VMEM((_BPW * 3,), jnp.float32),
            pltpu.VMEM((_BPW * 3,), jnp.float32),
            pltpu.VMEM((_BPW,), jnp.float32),
            pltpu.VMEM((_BPW * 2,), jnp.float32),
            pltpu.SemaphoreType.DMA,
            pltpu.SemaphoreType.DMA,
        ],
    )
    rT, tT, ff, pT = run(rotvecs.T.reshape(-1), translations.T.reshape(-1),
                         f, pp.T.reshape(-1), idxf)
    r = rT.reshape(3, _B).T
    t = tT.reshape(3, _B).T
    pps = pT.reshape(2, _B).T
    return (r, t, ff, pps)
